# baseline (speedup 1.0000x reference)
import functools

import jax
import jax.numpy as jnp
from jax import lax
from jax.experimental import pallas as pl
from jax.experimental.pallas import tpu as pltpu
from jax.experimental.pallas import tpu_sc as plsc


def _make_sc_kernel(B, F, D, NC, NS):
    NW = NC * NS
    CB = 1024
    n_bc = B // CB
    n_units = F * n_bc
    u_per_w = n_units // NW
    assert n_units % NW == 0
    BT = B // 128
    HALF = 8 * CB

    mesh = plsc.VectorSubcoreMesh(core_axis_name="c", subcore_axis_name="s")

    @functools.partial(
        pl.kernel,
        mesh=mesh,
        compiler_params=pltpu.CompilerParams(
            use_tc_tiling_on_sc=False, needs_layout_passes=False),
        out_type=jax.ShapeDtypeStruct((F, 2, 16 * B // 2), jnp.float32),
        scratch_types=[
            pltpu.VMEM((2, CB), jnp.int32),
            pltpu.VMEM((2, CB, D), jnp.float32),
            pltpu.VMEM((16512,), jnp.float32),
            pltpu.VMEM((2 * 2 * HALF,), jnp.float32),
            pltpu.VMEM((D,), jnp.float32),
            pltpu.SemaphoreType.DMA,
            pltpu.SemaphoreType.DMA,
            pltpu.SemaphoreType.DMA,
            pltpu.SemaphoreType.DMA,
            pltpu.SemaphoreType.DMA,
        ],
    )
    def run(xt_hbm, w_hbm, t_hbm, out_hbm,
            idx_v, buf_v, pad_v, obuf_v, t_v, sg0, sg1, so0, so1, st):
        wid = lax.axis_index("s") * NC + lax.axis_index("c")

        pltpu.sync_copy(t_hbm, t_v)
        t = t_v[...]
        thr = 1.0 / (1.0 + jnp.exp(-t))

        lanes = lax.iota(jnp.int32, 16)
        pat = (lanes >> 3) * 1032 + (lanes & 7) * 129

        def unit_fc(k):
            u = wid * u_per_w + k
            return u // n_bc, u % n_bc

        def stage_idx(k, slot):
            f, c = unit_fc(k)
            pltpu.sync_copy(xt_hbm.at[f, pl.ds(c * CB, CB)], idx_v.at[slot])

        sg = (sg0, sg1)
        so = (so0, so1)

        def start_gather(slot):
            return pltpu.async_copy(
                w_hbm.at[idx_v.at[slot]], buf_v.at[slot], sg[slot])

        def compute(slot):
            def row(i, carry):
                v = buf_v[slot, i, :]
                r = jnp.maximum(jnp.abs(v) - thr, 0.0)
                vu = plsc.bitcast(v, jnp.uint32)
                ru = plsc.bitcast(r, jnp.uint32)
                o = plsc.bitcast(ru | (vu & jnp.uint32(0x80000000)),
                                 jnp.float32)
                base = i + (i >> 7) * 1936
                bvec = lax.broadcast_in_dim(base, (16,), ())
                plsc.store_scatter(pad_v, [pat + bvec], o)
                return carry
            lax.fori_loop(0, CB, row, 0, unroll=16)

            def chunk(w, carry):
                ti = w >> 9
                tj = (w >> 6) & 7
                di = (w >> 3) & 7
                c = w & 7
                src = tj * 2064 + ti * 1032 + di * 129 + c * 16
                svec = lax.broadcast_in_dim(src, (16,), ())
                g = plsc.load_gather(pad_v, [svec + lanes])
                obuf_v[pl.ds(slot * 2 * HALF + w * 16, 16)] = g
                return carry
            lax.fori_loop(0, 2 * HALF // 16, chunk, 0, unroll=16)

        def start_out(k, slot):
            f, c = unit_fc(k)
            return [pltpu.async_copy(
                obuf_v.at[pl.ds((slot * 2 + ti) * HALF, HALF)],
                out_hbm.at[f, ti, pl.ds(c * HALF, HALF)], so[slot])
                for ti in range(2)]

        outs = [None, None]
        stage_idx(0, 0)
        gathers = [start_gather(0), None]
        for k in range(u_per_w):
            slot = k % 2
            if k + 1 < u_per_w:
                stage_idx(k + 1, 1 - slot)
                gathers[1 - slot] = start_gather(1 - slot)
            gathers[slot].wait()
            if outs[slot] is not None:
                for o in outs[slot]:
                    o.wait()
            compute(slot)
            outs[slot] = start_out(k, slot)
        for pair in outs:
            if pair is not None:
                for o in pair:
                    o.wait()

    return run


def kernel(x, weight, threshold):
    B, F = x.shape
    V, D = weight.shape
    info = plsc.get_sparse_core_info()
    run = _make_sc_kernel(B, F, D, info.num_cores, info.num_subcores)
    xt = x.T.astype(jnp.int32)
    out3 = run(xt, weight, threshold)
    out5 = out3.reshape(F, 2, B // 128, 8, 128)
    return out5.transpose(2, 4, 0, 1, 3).reshape(B, F, D)


# speedup vs baseline: 1.2271x; 1.2271x over previous
import functools

import jax
import jax.numpy as jnp
from jax import lax
from jax.experimental import pallas as pl
from jax.experimental.pallas import tpu as pltpu
from jax.experimental.pallas import tpu_sc as plsc


def _make_sc_kernel(B, F, D, NC, NS):
    NW = NC * NS
    CB = 1024
    n_bc = B // CB
    n_units = F * n_bc
    u_per_w = n_units // NW
    assert n_units % NW == 0
    BT = B // 128
    HALF = 8 * CB

    mesh = plsc.VectorSubcoreMesh(core_axis_name="c", subcore_axis_name="s")

    @functools.partial(
        pl.kernel,
        mesh=mesh,
        compiler_params=pltpu.CompilerParams(
            use_tc_tiling_on_sc=False, needs_layout_passes=False),
        out_type=jax.ShapeDtypeStruct((F, 2, 16 * B // 2), jnp.float32),
        scratch_types=[
            pltpu.VMEM((2, CB), jnp.int32),
            pltpu.VMEM((2, CB, D), jnp.float32),
            pltpu.VMEM((16512,), jnp.float32),
            pltpu.VMEM((2 * 2 * HALF,), jnp.float32),
            pltpu.VMEM((D,), jnp.float32),
            pltpu.SemaphoreType.DMA,
            pltpu.SemaphoreType.DMA,
            pltpu.SemaphoreType.DMA,
            pltpu.SemaphoreType.DMA,
            pltpu.SemaphoreType.DMA,
        ],
    )
    def run(xt_hbm, w_hbm, t_hbm, out_hbm,
            idx_v, buf_v, pad_v, obuf_v, t_v, sg0, sg1, so0, so1, st):
        wid = lax.axis_index("s") * NC + lax.axis_index("c")

        pltpu.sync_copy(t_hbm, t_v)
        t = t_v[...]
        thr = 1.0 / (1.0 + jnp.exp(-t))

        lanes = lax.iota(jnp.int32, 16)
        pat = (lanes >> 3) * 1032 + (lanes & 7) * 129

        def unit_fc(k):
            u = wid * u_per_w + k
            return u // n_bc, u % n_bc

        def stage_idx(k, slot):
            f, c = unit_fc(k)
            pltpu.sync_copy(xt_hbm.at[f, pl.ds(c * CB, CB)], idx_v.at[slot])

        sg = (sg0, sg1)
        so = (so0, so1)

        def start_gather(slot):
            return pltpu.async_copy(
                w_hbm.at[idx_v.at[slot]], buf_v.at[slot], sg[slot])

        def compute(slot):
            @plsc.parallel_loop(0, CB, unroll=8)
            def row(i):
                v = buf_v[slot, i, :]
                r = jnp.maximum(jnp.abs(v) - thr, 0.0)
                vu = plsc.bitcast(v, jnp.uint32)
                ru = plsc.bitcast(r, jnp.uint32)
                o = plsc.bitcast(ru | (vu & jnp.uint32(0x80000000)),
                                 jnp.float32)
                base = i + (i >> 7) * 1936
                bvec = lax.broadcast_in_dim(base, (16,), ())
                plsc.store_scatter(pad_v, [pat + bvec], o)

            @plsc.parallel_loop(0, 2 * HALF // 16, unroll=8)
            def chunk(w):
                ti = w >> 9
                tj = (w >> 6) & 7
                di = (w >> 3) & 7
                c = w & 7
                src = tj * 2064 + ti * 1032 + di * 129 + c * 16
                svec = lax.broadcast_in_dim(src, (16,), ())
                g = plsc.load_gather(pad_v, [svec + lanes])
                obuf_v[pl.ds(slot * 2 * HALF + w * 16, 16)] = g

        def start_out(k, slot):
            f, c = unit_fc(k)
            return [pltpu.async_copy(
                obuf_v.at[pl.ds((slot * 2 + ti) * HALF, HALF)],
                out_hbm.at[f, ti, pl.ds(c * HALF, HALF)], so[slot])
                for ti in range(2)]

        outs = [None, None]
        stage_idx(0, 0)
        gathers = [start_gather(0), None]
        for k in range(u_per_w):
            slot = k % 2
            if k + 1 < u_per_w:
                stage_idx(k + 1, 1 - slot)
                gathers[1 - slot] = start_gather(1 - slot)
            gathers[slot].wait()
            if outs[slot] is not None:
                for o in outs[slot]:
                    o.wait()
            compute(slot)
            outs[slot] = start_out(k, slot)
        for pair in outs:
            if pair is not None:
                for o in pair:
                    o.wait()

    return run


def kernel(x, weight, threshold):
    B, F = x.shape
    V, D = weight.shape
    info = plsc.get_sparse_core_info()
    run = _make_sc_kernel(B, F, D, info.num_cores, info.num_subcores)
    xt = x.T.astype(jnp.int32)
    out3 = run(xt, weight, threshold)
    out5 = out3.reshape(F, 2, B // 128, 8, 128)
    return out5.transpose(2, 4, 0, 1, 3).reshape(B, F, D)


# speedup vs baseline: 1.2301x; 1.0025x over previous
import functools

import jax
import jax.numpy as jnp
from jax import lax
from jax.experimental import pallas as pl
from jax.experimental.pallas import tpu as pltpu
from jax.experimental.pallas import tpu_sc as plsc


def _make_sc_kernel(B, F, D, NC, NS):
    NW = NC * NS
    CB = 1024
    n_bc = B // CB
    n_units = F * n_bc
    u_per_w = n_units // NW
    assert n_units % NW == 0
    BT = B // 128
    HALF = 8 * CB

    mesh = plsc.VectorSubcoreMesh(core_axis_name="c", subcore_axis_name="s")

    @functools.partial(
        pl.kernel,
        mesh=mesh,
        compiler_params=pltpu.CompilerParams(
            use_tc_tiling_on_sc=False, needs_layout_passes=False),
        out_type=jax.ShapeDtypeStruct((F, 2, 16 * B // 2), jnp.float32),
        scratch_types=[
            pltpu.VMEM((2, CB), jnp.int32),
            pltpu.VMEM((2, CB, D), jnp.float32),
            pltpu.VMEM((16512,), jnp.float32),
            pltpu.VMEM((2 * 2 * HALF,), jnp.float32),
            pltpu.VMEM((D,), jnp.float32),
            pltpu.SemaphoreType.DMA,
            pltpu.SemaphoreType.DMA,
            pltpu.SemaphoreType.DMA,
            pltpu.SemaphoreType.DMA,
            pltpu.SemaphoreType.DMA,
        ],
    )
    def run(xt_hbm, w_hbm, t_hbm, out_hbm,
            idx_v, buf_v, pad_v, obuf_v, t_v, sg0, sg1, so0, so1, st):
        wid = lax.axis_index("s") * NC + lax.axis_index("c")

        pltpu.sync_copy(t_hbm, t_v)
        t = t_v[...]
        thr = 1.0 / (1.0 + jnp.exp(-t))

        lanes = lax.iota(jnp.int32, 16)
        pat = (lanes >> 3) * 1032 + (lanes & 7) * 129

        def unit_fc(k):
            u = wid * u_per_w + k
            return u // n_bc, u % n_bc

        def stage_idx(k, slot):
            f, c = unit_fc(k)
            pltpu.sync_copy(xt_hbm.at[f, pl.ds(c * CB, CB)], idx_v.at[slot])

        sg = (sg0, sg1)
        so = (so0, so1)

        def start_gather(slot):
            return pltpu.async_copy(
                w_hbm.at[idx_v.at[slot]], buf_v.at[slot], sg[slot])

        def compute(slot):
            @plsc.parallel_loop(0, CB, unroll=16)
            def row(i):
                v = buf_v[slot, i, :]
                r = jnp.maximum(jnp.abs(v) - thr, 0.0)
                vu = plsc.bitcast(v, jnp.uint32)
                ru = plsc.bitcast(r, jnp.uint32)
                o = plsc.bitcast(ru | (vu & jnp.uint32(0x80000000)),
                                 jnp.float32)
                base = i + (i >> 7) * 1936
                bvec = lax.broadcast_in_dim(base, (16,), ())
                plsc.store_scatter(pad_v, [pat + bvec], o)

            @plsc.parallel_loop(0, 2 * HALF // 16, unroll=16)
            def chunk(w):
                ti = w >> 9
                tj = (w >> 6) & 7
                di = (w >> 3) & 7
                c = w & 7
                src = tj * 2064 + ti * 1032 + di * 129 + c * 16
                svec = lax.broadcast_in_dim(src, (16,), ())
                g = plsc.load_gather(pad_v, [svec + lanes])
                obuf_v[pl.ds(slot * 2 * HALF + w * 16, 16)] = g

        def start_out(k, slot):
            f, c = unit_fc(k)
            return [pltpu.async_copy(
                obuf_v.at[pl.ds((slot * 2 + ti) * HALF, HALF)],
                out_hbm.at[f, ti, pl.ds(c * HALF, HALF)], so[slot])
                for ti in range(2)]

        outs = [None, None]
        stage_idx(0, 0)
        gathers = [start_gather(0), None]
        for k in range(u_per_w):
            slot = k % 2
            if k + 1 < u_per_w:
                stage_idx(k + 1, 1 - slot)
                gathers[1 - slot] = start_gather(1 - slot)
            gathers[slot].wait()
            if outs[slot] is not None:
                for o in outs[slot]:
                    o.wait()
            compute(slot)
            outs[slot] = start_out(k, slot)
        for pair in outs:
            if pair is not None:
                for o in pair:
                    o.wait()

    return run


def kernel(x, weight, threshold):
    B, F = x.shape
    V, D = weight.shape
    info = plsc.get_sparse_core_info()
    run = _make_sc_kernel(B, F, D, info.num_cores, info.num_subcores)
    xt = x.T.astype(jnp.int32)
    out3 = run(xt, weight, threshold)
    out5 = out3.reshape(F, 2, B // 128, 8, 128)
    return out5.transpose(2, 4, 0, 1, 3).reshape(B, F, D)
